# baseline (speedup 1.0000x reference)
import functools

import jax
import jax.numpy as jnp
from jax import lax
from jax.experimental import pallas as pl
from jax.experimental.pallas import tpu as pltpu
from jax.experimental.pallas import tpu_sc as plsc

_NC = 2
_NS = 16
_NW = _NC * _NS
_K = 128
_G = 16


@functools.lru_cache(maxsize=None)
def _build_sc_agg(N_pad, D, NG, with_cnt):
    RPT = N_pad // _NS
    mesh = plsc.VectorSubcoreMesh(
        core_axis_name="c", subcore_axis_name="s",
        num_cores=_NC, num_subcores=_NS)

    def body(h_hbm, grp_hbm, zrows_hbm, zcnt_hbm,
             agg_out, cnt_out,
             idx0, idx1, rows_a, rows_b, ones_v, agg_sh, cnt_sh,
             sem_a, sem_b, sem_i):
        c = lax.axis_index("c")
        s = lax.axis_index("s")
        wid = c * _NS + s

        pltpu.sync_copy(zrows_hbm, agg_sh.at[pl.ds(s * RPT, RPT)])

        if with_cnt:
            @pl.when(s == 0)
            def _():
                pltpu.sync_copy(zcnt_hbm, cnt_sh)

            for i in range(_K // 16):
                ones_v[pl.ds(i * 16, 16)] = jnp.ones((16,), jnp.float32)

        pltpu.sync_copy(grp_hbm.at[wid, 0], idx0)

        plsc.subcore_barrier()

        def gather(I, j, buf, sem):
            pltpu.async_copy(h_hbm.at[I.at[j]], buf, sem)

        def wait_g(buf, sem):
            pltpu.make_async_copy(h_hbm.at[idx0.at[0]], buf, sem).wait()

        def scat(I, j, buf):
            pltpu.sync_copy(buf, agg_sh.at[I.at[_G + j]], add=True)
            if with_cnt:
                pltpu.sync_copy(ones_v, cnt_sh.at[I.at[_G + j]], add=True)

        bufs = (idx0, idx1)
        for g in range(NG):
            I = bufs[g % 2]
            if g + 1 < NG:
                pltpu.async_copy(grp_hbm.at[wid, g + 1], bufs[(g + 1) % 2],
                                 sem_i)

            @pl.loop(0, _G)
            def _(j):
                gather(I, j, rows_a, sem_a)
                wait_g(rows_a, sem_a)
                scat(I, j, rows_a)

            if g + 1 < NG:
                pltpu.make_async_copy(grp_hbm.at[wid, 0], bufs[(g + 1) % 2],
                                      sem_i).wait()

        plsc.subcore_barrier()

        pltpu.sync_copy(agg_sh.at[pl.ds(s * RPT, RPT)], agg_out.at[wid])

        if with_cnt:
            @pl.when(s == 0)
            def _():
                pltpu.sync_copy(cnt_sh, cnt_out.at[c])

    out_type = [jax.ShapeDtypeStruct((_NW, RPT, D), jnp.float32)]
    if with_cnt:
        out_type.append(jax.ShapeDtypeStruct((_NC, N_pad), jnp.float32))

    def wrapped(h, grp, zrows, zcnt):
        def body_in(*args):
            if with_cnt:
                (h_hbm, grp_hbm, zrows_hbm, zcnt_hbm,
                 agg_out, cnt_out, *rest) = args
            else:
                (h_hbm, grp_hbm, zrows_hbm, zcnt_hbm,
                 agg_out, *rest) = args
                cnt_out = None
            body(h_hbm, grp_hbm, zrows_hbm, zcnt_hbm,
                 agg_out, cnt_out, *rest)

        return pl.kernel(
            body_in,
            out_type=out_type,
            mesh=mesh,
            scratch_types=[
                pltpu.VMEM((2 * _G, _K), jnp.int32),
                pltpu.VMEM((2 * _G, _K), jnp.int32),
                pltpu.VMEM((_K, D), jnp.float32),
                pltpu.VMEM((_K, D), jnp.float32),
                pltpu.VMEM((_K,), jnp.float32),
                pltpu.VMEM_SHARED((N_pad, D), jnp.float32),
                pltpu.VMEM_SHARED((N_pad,), jnp.float32),
                pltpu.SemaphoreType.DMA,
                pltpu.SemaphoreType.DMA,
                pltpu.SemaphoreType.DMA,
            ],
        )(h, grp, zrows, zcnt)

    return wrapped


def _dot(a, b):
    return jnp.dot(a, b, precision=lax.Precision.HIGHEST,
                   preferred_element_type=jnp.float32)


def _tc_layer_call(agg, cntT, h, Wl, bl, Wr, residual, head=None):
    N, D = h.shape
    BN = 1024
    grid = (pl.cdiv(N, BN),)

    def body(agg_ref, cnt_ref, h_ref, Wl_ref, bl_ref, Wr_ref, *rest):
        out_ref = rest[-1]
        cnt = cnt_ref[..., 0:1] + cnt_ref[..., 1:2]
        invc = 1.0 / jnp.maximum(cnt, 1.0)
        mean = (agg_ref[0] + agg_ref[1]) * invc
        h_blk = h_ref[...]
        y = _dot(mean, Wl_ref[...]) + bl_ref[...] + _dot(h_blk, Wr_ref[...])
        y = jnp.maximum(y, 0.0)
        if residual:
            y = y + h_blk
        if head is not None:
            Wh1_ref, bh1_ref, Wh2_ref, bh2_ref = rest[:4]
            t = jnp.maximum(_dot(y, Wh1_ref[...]) + bh1_ref[...], 0.0)
            y = _dot(t, Wh2_ref[...]) + bh2_ref[...]
        out_ref[...] = y

    w_spec = pl.BlockSpec((D, D), lambda i: (0, 0))
    b_spec = pl.BlockSpec((1, D), lambda i: (0, 0))
    in_specs = [
        pl.BlockSpec((_NC, BN, D), lambda i: (0, i, 0)),
        pl.BlockSpec((BN, _NC), lambda i: (i, 0)),
        pl.BlockSpec((BN, D), lambda i: (i, 0)),
        w_spec, b_spec, w_spec,
    ]
    args = [agg, cntT, h, Wl, bl.reshape(1, D), Wr]
    if head is not None:
        Wh1, bh1, Wh2, bh2 = head
        in_specs += [w_spec, b_spec, w_spec, b_spec]
        args += [Wh1, bh1.reshape(1, D), Wh2, bh2.reshape(1, D)]

    return pl.pallas_call(
        body,
        grid=grid,
        in_specs=in_specs,
        out_specs=pl.BlockSpec((BN, D), lambda i: (i, 0)),
        out_shape=jax.ShapeDtypeStruct((N, D), jnp.float32),
    )(*args)


def kernel(x, edge_index, Wl0, bl0, Wr0, Wl1, bl1, Wr1, Wl2, bl2, Wr2,
           Wh1, bh1, Wh2, bh2):
    N, D = x.shape
    E = edge_index.shape[1]
    NG = pl.cdiv(E, _NW * _K * _G)
    E_pad = _NW * _K * _G * NG
    N_pad = pl.cdiv(N + 1, 128) * 128
    RPT = N_pad // _NS

    src = edge_index[0]
    dst = edge_index[1]
    pad = E_pad - E
    srcp = jnp.concatenate(
        [src, jnp.zeros((pad,), jnp.int32)]).reshape(_NW, NG, _G, _K)
    dstp = jnp.concatenate(
        [dst, jnp.full((pad,), N, jnp.int32)]).reshape(_NW, NG, _G, _K)
    grp = jnp.concatenate([srcp, dstp], axis=2)
    zrows = jnp.zeros((RPT, D), jnp.float32)
    zcnt = jnp.zeros((N_pad,), jnp.float32)

    sc_agg0 = _build_sc_agg(N_pad, D, NG, True)
    layers = [(Wl0, bl0, Wr0), (Wl1, bl1, Wr1), (Wl2, bl2, Wr2)]

    h = x
    cntT = None
    for i, (Wl, bl, Wr) in enumerate(layers):
        agg_raw, cnt_raw = sc_agg0(h, grp, zrows, zcnt)
        if cntT is None:
            cntT = cnt_raw.T
        agg = agg_raw.reshape(_NC, N_pad, D)
        h = _tc_layer_call(agg, cntT, h, Wl, bl, Wr,
                           residual=(i > 0),
                           head=(Wh1, bh1, Wh2, bh2) if i == 2 else None)
    return h


# speedup vs baseline: 2.6371x; 2.6371x over previous
import functools

import jax
import jax.numpy as jnp
from jax import lax
from jax.experimental import pallas as pl
from jax.experimental.pallas import tpu as pltpu
from jax.experimental.pallas import tpu_sc as plsc

_NC = 2
_NS = 16
_NW = _NC * _NS
_K = 128
_G = 16


@functools.lru_cache(maxsize=None)
def _build_sc_agg(N_pad, D, C, with_cnt):
    RPT = N_pad // _NS
    mesh = plsc.VectorSubcoreMesh(
        core_axis_name="c", subcore_axis_name="s",
        num_cores=_NC, num_subcores=_NS)

    def body(h_hbm, src_hbm, dst_hbm, zrows_hbm, zcnt_hbm,
             agg_out, cnt_out,
             src_v, dst_v, rows_v, ones_v, agg_sh, cnt_sh, sem):
        c = lax.axis_index("c")
        s = lax.axis_index("s")
        wid = c * _NS + s

        pltpu.sync_copy(zrows_hbm, agg_sh.at[pl.ds(s * RPT, RPT)])

        if with_cnt:
            @pl.when(s == 0)
            def _():
                pltpu.sync_copy(zcnt_hbm, cnt_sh)

            for i in range(_K // 16):
                ones_v[pl.ds(i * 16, 16)] = jnp.ones((16,), jnp.float32)

        pltpu.sync_copy(src_hbm.at[wid], src_v)
        pltpu.sync_copy(dst_hbm.at[wid], dst_v)

        plsc.subcore_barrier()

        @pl.loop(0, C)
        def _(j):
            pltpu.async_copy(h_hbm.at[src_v.at[j]], rows_v, sem).wait()
            pltpu.sync_copy(rows_v, agg_sh.at[dst_v.at[j]], add=True)
            if with_cnt:
                pltpu.sync_copy(ones_v, cnt_sh.at[dst_v.at[j]], add=True)

        plsc.subcore_barrier()

        pltpu.sync_copy(agg_sh.at[pl.ds(s * RPT, RPT)], agg_out.at[wid])

        if with_cnt:
            @pl.when(s == 0)
            def _():
                pltpu.sync_copy(cnt_sh, cnt_out.at[c])

    out_type = [jax.ShapeDtypeStruct((_NW, RPT, D), jnp.float32)]
    if with_cnt:
        out_type.append(jax.ShapeDtypeStruct((_NC, N_pad), jnp.float32))

    def wrapped(h, srcp, dstp, zrows, zcnt):
        def body_in(*args):
            if with_cnt:
                (h_hbm, src_hbm, dst_hbm, zrows_hbm, zcnt_hbm,
                 agg_out, cnt_out, *rest) = args
            else:
                (h_hbm, src_hbm, dst_hbm, zrows_hbm, zcnt_hbm,
                 agg_out, *rest) = args
                cnt_out = None
            body(h_hbm, src_hbm, dst_hbm, zrows_hbm, zcnt_hbm,
                 agg_out, cnt_out, *rest)

        return pl.kernel(
            body_in,
            out_type=out_type,
            mesh=mesh,
            scratch_types=[
                pltpu.VMEM((C, _K), jnp.int32),
                pltpu.VMEM((C, _K), jnp.int32),
                pltpu.VMEM((_K, D), jnp.float32),
                pltpu.VMEM((_K,), jnp.float32),
                pltpu.VMEM_SHARED((N_pad, D), jnp.float32),
                pltpu.VMEM_SHARED((N_pad,), jnp.float32),
                pltpu.SemaphoreType.DMA,
            ],
        )(h, srcp, dstp, zrows, zcnt)

    return wrapped


def _dot(a, b):
    return jnp.dot(a, b, precision=lax.Precision.HIGHEST,
                   preferred_element_type=jnp.float32)


def _tc_layer_call(agg, cntT, h, Wl, bl, Wr, residual, head=None):
    N, D = h.shape
    BN = 1024
    grid = (pl.cdiv(N, BN),)

    def body(agg_ref, cnt_ref, h_ref, Wl_ref, bl_ref, Wr_ref, *rest):
        out_ref = rest[-1]
        cnt = cnt_ref[..., 0:1] + cnt_ref[..., 1:2]
        invc = 1.0 / jnp.maximum(cnt, 1.0)
        mean = (agg_ref[0] + agg_ref[1]) * invc
        h_blk = h_ref[...]
        y = _dot(mean, Wl_ref[...]) + bl_ref[...] + _dot(h_blk, Wr_ref[...])
        y = jnp.maximum(y, 0.0)
        if residual:
            y = y + h_blk
        if head is not None:
            Wh1_ref, bh1_ref, Wh2_ref, bh2_ref = rest[:4]
            t = jnp.maximum(_dot(y, Wh1_ref[...]) + bh1_ref[...], 0.0)
            y = _dot(t, Wh2_ref[...]) + bh2_ref[...]
        out_ref[...] = y

    w_spec = pl.BlockSpec((D, D), lambda i: (0, 0))
    b_spec = pl.BlockSpec((1, D), lambda i: (0, 0))
    in_specs = [
        pl.BlockSpec((_NC, BN, D), lambda i: (0, i, 0)),
        pl.BlockSpec((BN, _NC), lambda i: (i, 0)),
        pl.BlockSpec((BN, D), lambda i: (i, 0)),
        w_spec, b_spec, w_spec,
    ]
    args = [agg, cntT, h, Wl, bl.reshape(1, D), Wr]
    if head is not None:
        Wh1, bh1, Wh2, bh2 = head
        in_specs += [w_spec, b_spec, w_spec, b_spec]
        args += [Wh1, bh1.reshape(1, D), Wh2, bh2.reshape(1, D)]

    return pl.pallas_call(
        body,
        grid=grid,
        in_specs=in_specs,
        out_specs=pl.BlockSpec((BN, D), lambda i: (i, 0)),
        out_shape=jax.ShapeDtypeStruct((N, D), jnp.float32),
    )(*args)


def kernel(x, edge_index, Wl0, bl0, Wr0, Wl1, bl1, Wr1, Wl2, bl2, Wr2,
           Wh1, bh1, Wh2, bh2):
    N, D = x.shape
    E = edge_index.shape[1]
    C = pl.cdiv(E, _NW * _K)
    E_pad = _NW * _K * C
    N_pad = pl.cdiv(N + 1, 128) * 128
    RPT = N_pad // _NS

    src = edge_index[0]
    dst = edge_index[1]
    pad = E_pad - E
    pad_ar = jnp.arange(pad, dtype=jnp.int32)
    pad_src = pad_ar % jnp.int32(N)
    pad_dst = jnp.int32(N) + pad_ar % jnp.int32(N_pad - N)
    srcp = jnp.concatenate([src, pad_src]).reshape(_NW, C, _K)
    dstp = jnp.concatenate([dst, pad_dst]).reshape(_NW, C, _K)
    zrows = jnp.zeros((RPT, D), jnp.float32)
    zcnt = jnp.zeros((N_pad,), jnp.float32)

    sc_agg0 = _build_sc_agg(N_pad, D, C, True)
    layers = [(Wl0, bl0, Wr0), (Wl1, bl1, Wr1), (Wl2, bl2, Wr2)]

    h = x
    cntT = None
    for i, (Wl, bl, Wr) in enumerate(layers):
        agg_raw, cnt_raw = sc_agg0(h, srcp, dstp, zrows, zcnt)
        if cntT is None:
            cntT = cnt_raw.T
        agg = agg_raw.reshape(_NC, N_pad, D)
        h = _tc_layer_call(agg, cntT, h, Wl, bl, Wr,
                           residual=(i > 0),
                           head=(Wh1, bh1, Wh2, bh2) if i == 2 else None)
    return h


# speedup vs baseline: 2.7082x; 1.0270x over previous
import functools

import jax
import jax.numpy as jnp
from jax import lax
from jax.experimental import pallas as pl
from jax.experimental.pallas import tpu as pltpu
from jax.experimental.pallas import tpu_sc as plsc

_NC = 2
_NS = 16
_NW = _NC * _NS
_K = 128
_G = 16


@functools.lru_cache(maxsize=None)
def _build_sc_agg(N_pad, D, C, with_cnt):
    RPT = N_pad // _NS
    mesh = plsc.VectorSubcoreMesh(
        core_axis_name="c", subcore_axis_name="s",
        num_cores=_NC, num_subcores=_NS)

    def body(h_hbm, src_hbm, dst_hbm, zrows_hbm, zcnt_hbm,
             agg_out, cnt_out,
             src_v, dst_v, rows_v, ones_v, agg_sh, cnt_sh, sem):
        c = lax.axis_index("c")
        s = lax.axis_index("s")
        wid = c * _NS + s

        pltpu.sync_copy(zrows_hbm, agg_sh.at[pl.ds(s * RPT, RPT)])

        if with_cnt:
            @pl.when(s == 0)
            def _():
                pltpu.sync_copy(zcnt_hbm, cnt_sh)

            for i in range(_K // 16):
                ones_v[pl.ds(i * 16, 16)] = jnp.ones((16,), jnp.float32)

        pltpu.sync_copy(src_hbm.at[wid], src_v)
        pltpu.sync_copy(dst_hbm.at[wid], dst_v)

        plsc.subcore_barrier()

        @pl.loop(0, C)
        def _(j):
            pltpu.async_copy(h_hbm.at[src_v.at[j]], rows_v, sem).wait()
            pltpu.sync_copy(rows_v, agg_sh.at[dst_v.at[j]], add=True)
            if with_cnt:
                pltpu.sync_copy(ones_v, cnt_sh.at[dst_v.at[j]], add=True)

        plsc.subcore_barrier()

        pltpu.sync_copy(agg_sh.at[pl.ds(s * RPT, RPT)], agg_out.at[wid])

        if with_cnt:
            @pl.when(s == 0)
            def _():
                pltpu.sync_copy(cnt_sh, cnt_out.at[c])

    out_type = [jax.ShapeDtypeStruct((_NW, RPT, D), jnp.float32)]
    if with_cnt:
        out_type.append(jax.ShapeDtypeStruct((_NC, N_pad), jnp.float32))

    def wrapped(h, srcp, dstp, zrows, zcnt):
        def body_in(*args):
            if with_cnt:
                (h_hbm, src_hbm, dst_hbm, zrows_hbm, zcnt_hbm,
                 agg_out, cnt_out, *rest) = args
            else:
                (h_hbm, src_hbm, dst_hbm, zrows_hbm, zcnt_hbm,
                 agg_out, *rest) = args
                cnt_out = None
            body(h_hbm, src_hbm, dst_hbm, zrows_hbm, zcnt_hbm,
                 agg_out, cnt_out, *rest)

        return pl.kernel(
            body_in,
            out_type=out_type,
            mesh=mesh,
            scratch_types=[
                pltpu.VMEM((C, _K), jnp.int32),
                pltpu.VMEM((C, _K), jnp.int32),
                pltpu.VMEM((_K, D), jnp.float32),
                pltpu.VMEM((_K,), jnp.float32),
                pltpu.VMEM_SHARED((N_pad, D), jnp.float32),
                pltpu.VMEM_SHARED((N_pad,), jnp.float32),
                pltpu.SemaphoreType.DMA,
            ],
        )(h, srcp, dstp, zrows, zcnt)

    return wrapped


def _dot(a, b):
    return jnp.dot(a, b, precision=lax.Precision.HIGHEST,
                   preferred_element_type=jnp.float32)


def _tc_layer_call(agg, cntT, h, Wl, bl, Wr, residual, head=None):
    N, D = h.shape
    BN = 1024
    grid = (pl.cdiv(N, BN),)

    def body(agg_ref, cnt_ref, h_ref, Wl_ref, bl_ref, Wr_ref, *rest):
        out_ref = rest[-1]
        cnt = cnt_ref[..., 0:1] + cnt_ref[..., 1:2]
        invc = 1.0 / jnp.maximum(cnt, 1.0)
        mean = (agg_ref[0] + agg_ref[1]) * invc
        h_blk = h_ref[...]
        y = _dot(mean, Wl_ref[...]) + bl_ref[...] + _dot(h_blk, Wr_ref[...])
        y = jnp.maximum(y, 0.0)
        if residual:
            y = y + h_blk
        if head is not None:
            Wh1_ref, bh1_ref, Wh2_ref, bh2_ref = rest[:4]
            t = jnp.maximum(_dot(y, Wh1_ref[...]) + bh1_ref[...], 0.0)
            y = _dot(t, Wh2_ref[...]) + bh2_ref[...]
        out_ref[...] = y

    w_spec = pl.BlockSpec((D, D), lambda i: (0, 0))
    b_spec = pl.BlockSpec((1, D), lambda i: (0, 0))
    in_specs = [
        pl.BlockSpec((_NC, BN, D), lambda i: (0, i, 0)),
        pl.BlockSpec((BN, _NC), lambda i: (i, 0)),
        pl.BlockSpec((BN, D), lambda i: (i, 0)),
        w_spec, b_spec, w_spec,
    ]
    args = [agg, cntT, h, Wl, bl.reshape(1, D), Wr]
    if head is not None:
        Wh1, bh1, Wh2, bh2 = head
        in_specs += [w_spec, b_spec, w_spec, b_spec]
        args += [Wh1, bh1.reshape(1, D), Wh2, bh2.reshape(1, D)]

    return pl.pallas_call(
        body,
        grid=grid,
        in_specs=in_specs,
        out_specs=pl.BlockSpec((BN, D), lambda i: (i, 0)),
        out_shape=jax.ShapeDtypeStruct((N, D), jnp.float32),
    )(*args)


def kernel(x, edge_index, Wl0, bl0, Wr0, Wl1, bl1, Wr1, Wl2, bl2, Wr2,
           Wh1, bh1, Wh2, bh2):
    N, D = x.shape
    E = edge_index.shape[1]
    C = pl.cdiv(E, _NW * _K)
    E_pad = _NW * _K * C
    N_pad = pl.cdiv(N + 1, 128) * 128
    RPT = N_pad // _NS

    src = edge_index[0]
    dst = edge_index[1]
    pad = E_pad - E
    pad_ar = jnp.arange(pad, dtype=jnp.int32)
    pad_src = pad_ar % jnp.int32(N)
    pad_dst = jnp.int32(N) + pad_ar % jnp.int32(N_pad - N)
    srcp = jnp.concatenate([src, pad_src]).reshape(_NW, C, _K)
    dstp = jnp.concatenate([dst, pad_dst]).reshape(_NW, C, _K)
    zrows = jnp.zeros((RPT, D), jnp.float32)
    zcnt = jnp.zeros((N_pad,), jnp.float32)

    sc_agg0 = _build_sc_agg(N_pad, D, C, True)
    sc_agg = _build_sc_agg(N_pad, D, C, False)
    layers = [(Wl0, bl0, Wr0), (Wl1, bl1, Wr1), (Wl2, bl2, Wr2)]

    h = x
    cntT = None
    for i, (Wl, bl, Wr) in enumerate(layers):
        if i == 0:
            agg_raw, cnt_raw = sc_agg0(h, srcp, dstp, zrows, zcnt)
            cntT = cnt_raw.T
        else:
            (agg_raw,) = sc_agg(h, srcp, dstp, zrows, zcnt)
        agg = agg_raw.reshape(_NC, N_pad, D)
        h = _tc_layer_call(agg, cntT, h, Wl, bl, Wr,
                           residual=(i > 0),
                           head=(Wh1, bh1, Wh2, bh2) if i == 2 else None)
    return h


# speedup vs baseline: 3.9916x; 1.4739x over previous
import functools

import jax
import jax.numpy as jnp
from jax import lax
from jax.experimental import pallas as pl
from jax.experimental.pallas import tpu as pltpu
from jax.experimental.pallas import tpu_sc as plsc

_NC = 2
_NS = 16
_NW = _NC * _NS
_K = 128
_G = 16


@functools.lru_cache(maxsize=None)
def _build_sc_agg(N_pad, D, C, with_cnt):
    RPT = N_pad // _NS
    mesh = plsc.VectorSubcoreMesh(
        core_axis_name="c", subcore_axis_name="s",
        num_cores=_NC, num_subcores=_NS)

    def body(h_hbm, idx_hbm, zrows_hbm, zcnt_hbm,
             agg_out, cnt_out,
             idx_v, rows_a, rows_b, srcu_a, dstu_a, srcu_b, dstu_b,
             ones_v, agg_sh, cnt_sh, sem_a, sem_b):
        c = lax.axis_index("c")
        s = lax.axis_index("s")
        wid = c * _NS + s

        pltpu.sync_copy(zrows_hbm, agg_sh.at[pl.ds(s * RPT, RPT)])

        if with_cnt:
            @pl.when(s == 0)
            def _():
                pltpu.sync_copy(zcnt_hbm, cnt_sh)

            for i in range(_K // 16):
                ones_v[pl.ds(i * 16, 16)] = jnp.ones((16,), jnp.float32)

        pltpu.sync_copy(idx_hbm.at[wid], idx_v)

        plsc.subcore_barrier()

        def unpack(j, srcu, dstu):
            for i in range(_K // 16):
                p = idx_v[j, pl.ds(i * 16, 16)]
                srcu[pl.ds(i * 16, 16)] = p & jnp.int32(0x3FFF)
                dstu[pl.ds(i * 16, 16)] = p >> jnp.int32(14)

        def gather(srcu, buf, sem):
            pltpu.async_copy(h_hbm.at[srcu], buf, sem)

        def wait_g(buf, sem):
            pltpu.make_async_copy(h_hbm.at[srcu_a], buf, sem).wait()

        def scat(dstu, buf):
            pltpu.sync_copy(buf, agg_sh.at[dstu], add=True)
            if with_cnt:
                pltpu.sync_copy(ones_v, cnt_sh.at[dstu], add=True)

        unpack(0, srcu_a, dstu_a)
        gather(srcu_a, rows_a, sem_a)
        unpack(1, srcu_b, dstu_b)
        gather(srcu_b, rows_b, sem_b)

        @pl.loop(0, C - 2, step=2)
        def _(j):
            wait_g(rows_a, sem_a)
            scat(dstu_a, rows_a)
            unpack(j + 2, srcu_a, dstu_a)
            gather(srcu_a, rows_a, sem_a)
            wait_g(rows_b, sem_b)
            scat(dstu_b, rows_b)
            unpack(j + 3, srcu_b, dstu_b)
            gather(srcu_b, rows_b, sem_b)

        wait_g(rows_a, sem_a)
        scat(dstu_a, rows_a)
        wait_g(rows_b, sem_b)
        scat(dstu_b, rows_b)

        plsc.subcore_barrier()

        pltpu.sync_copy(agg_sh.at[pl.ds(s * RPT, RPT)], agg_out.at[wid])

        if with_cnt:
            @pl.when(s == 0)
            def _():
                pltpu.sync_copy(cnt_sh, cnt_out.at[c])

    out_type = [jax.ShapeDtypeStruct((_NW, RPT, D), jnp.float32)]
    if with_cnt:
        out_type.append(jax.ShapeDtypeStruct((_NC, N_pad), jnp.float32))

    def wrapped(h, idxp, zrows, zcnt):
        def body_in(*args):
            if with_cnt:
                (h_hbm, idx_hbm, zrows_hbm, zcnt_hbm,
                 agg_out, cnt_out, *rest) = args
            else:
                (h_hbm, idx_hbm, zrows_hbm, zcnt_hbm,
                 agg_out, *rest) = args
                cnt_out = None
            body(h_hbm, idx_hbm, zrows_hbm, zcnt_hbm,
                 agg_out, cnt_out, *rest)

        return pl.kernel(
            body_in,
            out_type=out_type,
            mesh=mesh,
            scratch_types=[
                pltpu.VMEM((C, _K), jnp.int32),
                pltpu.VMEM((_K, D), jnp.float32),
                pltpu.VMEM((_K, D), jnp.float32),
                pltpu.VMEM((_K,), jnp.int32),
                pltpu.VMEM((_K,), jnp.int32),
                pltpu.VMEM((_K,), jnp.int32),
                pltpu.VMEM((_K,), jnp.int32),
                pltpu.VMEM((_K,), jnp.float32),
                pltpu.VMEM_SHARED((N_pad, D), jnp.float32),
                pltpu.VMEM_SHARED((N_pad,), jnp.float32),
                pltpu.SemaphoreType.DMA,
                pltpu.SemaphoreType.DMA,
            ],
        )(h, idxp, zrows, zcnt)

    return wrapped


def _dot(a, b):
    return jnp.dot(a, b, precision=lax.Precision.HIGHEST,
                   preferred_element_type=jnp.float32)


def _tc_layer_call(agg, cntT, h, Wl, bl, Wr, residual, head=None):
    N, D = h.shape
    BN = 1024
    grid = (pl.cdiv(N, BN),)

    def body(agg_ref, cnt_ref, h_ref, Wl_ref, bl_ref, Wr_ref, *rest):
        out_ref = rest[-1]
        cnt = cnt_ref[..., 0:1] + cnt_ref[..., 1:2]
        invc = 1.0 / jnp.maximum(cnt, 1.0)
        mean = (agg_ref[0] + agg_ref[1]) * invc
        h_blk = h_ref[...]
        y = _dot(mean, Wl_ref[...]) + bl_ref[...] + _dot(h_blk, Wr_ref[...])
        y = jnp.maximum(y, 0.0)
        if residual:
            y = y + h_blk
        if head is not None:
            Wh1_ref, bh1_ref, Wh2_ref, bh2_ref = rest[:4]
            t = jnp.maximum(_dot(y, Wh1_ref[...]) + bh1_ref[...], 0.0)
            y = _dot(t, Wh2_ref[...]) + bh2_ref[...]
        out_ref[...] = y

    w_spec = pl.BlockSpec((D, D), lambda i: (0, 0))
    b_spec = pl.BlockSpec((1, D), lambda i: (0, 0))
    in_specs = [
        pl.BlockSpec((_NC, BN, D), lambda i: (0, i, 0)),
        pl.BlockSpec((BN, _NC), lambda i: (i, 0)),
        pl.BlockSpec((BN, D), lambda i: (i, 0)),
        w_spec, b_spec, w_spec,
    ]
    args = [agg, cntT, h, Wl, bl.reshape(1, D), Wr]
    if head is not None:
        Wh1, bh1, Wh2, bh2 = head
        in_specs += [w_spec, b_spec, w_spec, b_spec]
        args += [Wh1, bh1.reshape(1, D), Wh2, bh2.reshape(1, D)]

    return pl.pallas_call(
        body,
        grid=grid,
        in_specs=in_specs,
        out_specs=pl.BlockSpec((BN, D), lambda i: (i, 0)),
        out_shape=jax.ShapeDtypeStruct((N, D), jnp.float32),
    )(*args)


def kernel(x, edge_index, Wl0, bl0, Wr0, Wl1, bl1, Wr1, Wl2, bl2, Wr2,
           Wh1, bh1, Wh2, bh2):
    N, D = x.shape
    E = edge_index.shape[1]
    C = 2 * pl.cdiv(E, _NW * _K * 2)
    E_pad = _NW * _K * C
    N_pad = pl.cdiv(N + 1, 128) * 128
    RPT = N_pad // _NS

    src = edge_index[0]
    dst = edge_index[1]
    pad = E_pad - E
    pad_ar = jnp.arange(pad, dtype=jnp.int32)
    pad_src = pad_ar % jnp.int32(N)
    pad_dst = jnp.int32(N) + pad_ar % jnp.int32(N_pad - N)
    srcp = jnp.concatenate([src, pad_src])
    dstp = jnp.concatenate([dst, pad_dst])
    idxp = (srcp | (dstp << jnp.int32(14))).reshape(_NW, C, _K)
    zrows = jnp.zeros((RPT, D), jnp.float32)
    zcnt = jnp.zeros((N_pad,), jnp.float32)

    sc_agg0 = _build_sc_agg(N_pad, D, C, True)
    sc_agg = _build_sc_agg(N_pad, D, C, False)
    layers = [(Wl0, bl0, Wr0), (Wl1, bl1, Wr1), (Wl2, bl2, Wr2)]

    h = x
    cntT = None
    for i, (Wl, bl, Wr) in enumerate(layers):
        if i == 0:
            agg_raw, cnt_raw = sc_agg0(h, idxp, zrows, zcnt)
            cntT = cnt_raw.T
        else:
            (agg_raw,) = sc_agg(h, idxp, zrows, zcnt)
        agg = agg_raw.reshape(_NC, N_pad, D)
        h = _tc_layer_call(agg, cntT, h, Wl, bl, Wr,
                           residual=(i > 0),
                           head=(Wh1, bh1, Wh2, bh2) if i == 2 else None)
    return h
